# baseline (speedup 1.0000x reference)
import functools
import jax
import jax.numpy as jnp
from jax import lax
from jax.experimental import pallas as pl
from jax.experimental.pallas import tpu as pltpu
from jax.experimental.pallas import tpu_sc as plsc

NX = 768
NF = 768
NX8 = NX // 8
NBITS = 8

NC, NS = 2, 16
NW = NC * NS
FPW = NF // NW
GV = NX8 // 16

_MAA = 0x00AA00AA
_MCC = 0x0000CCCC
_M0F = 0x0F0F0F0F
_MF0 = -252645136


def _sc_decode_body(codes_hbm, scale_hbm, wq_hbm,
                    codes_v, scale_v, out_v, slut_v):
    wid = lax.axis_index("s") * NC + lax.axis_index("c")
    f_base = wid * FPW
    pltpu.sync_copy(scale_hbm.at[pl.ds(f_base, FPW)], scale_v)
    pltpu.sync_copy(codes_hbm.at[:, pl.ds(f_base, FPW), :], codes_v)

    iota = lax.broadcasted_iota(jnp.int32, (16,), 0)
    iota8 = iota * 8

    def fl_body(fl):
        f_abs = f_base + fl
        svvec = scale_v[fl, :]
        sv = [jnp.full((16,), svvec[i], jnp.float32) for i in range(NBITS)]
        lo = jnp.zeros((16,), jnp.float32)
        hi = jnp.zeros((16,), jnp.float32)
        for j in range(4):
            bit = (iota >> (3 - j)) & 1
            lo = lo + jnp.where(bit != 0, sv[4 + j], -sv[4 + j])
            hi = hi + jnp.where(bit != 0, sv[j], -sv[j])
        sbase = fl * 256
        for k in range(16):
            slut_v[pl.ds(sbase + k * 16, 16)] = lo + hi[k]

        flvec = jnp.full((16,), fl, jnp.int32)
        for gh in range(GV // 2):
            rows2 = []
            for gv in (2 * gh, 2 * gh + 1):
                gsl = pl.ds(gv * 16, 16)
                v = [codes_v[i, fl, gsl] for i in range(NBITS)]
                xw = (v[0] << 24) | (v[1] << 16) | (v[2] << 8) | v[3]
                yw = (v[4] << 24) | (v[5] << 16) | (v[6] << 8) | v[7]
                t = (xw ^ (xw >> 7)) & _MAA
                xw = xw ^ t ^ (t << 7)
                t = (yw ^ (yw >> 7)) & _MAA
                yw = yw ^ t ^ (t << 7)
                t = (xw ^ (xw >> 14)) & _MCC
                xw = xw ^ t ^ (t << 14)
                t = (yw ^ (yw >> 14)) & _MCC
                yw = yw ^ t ^ (t << 14)
                t = (xw & jnp.int32(_MF0)) | ((yw >> 4) & _M0F)
                yw = ((xw << 4) & jnp.int32(_MF0)) | (yw & _M0F)
                xw = t
                rows2.append([(xw >> 24) & 0xFF, (xw >> 16) & 0xFF,
                              (xw >> 8) & 0xFF, xw & 0xFF,
                              (yw >> 24) & 0xFF, (yw >> 16) & 0xFF,
                              (yw >> 8) & 0xFF, yw & 0xFF])
            vals2 = [[plsc.load_gather(slut_v, [rows2[h][p] + sbase])
                      for p in range(8)] for h in range(2)]
            for h in range(2):
                for p in range(8):
                    cidx = iota8 + (128 * (2 * gh + h) + p)
                    plsc.store_scatter(out_v, [flvec, cidx], vals2[h][p])

    plsc.parallel_loop(0, FPW, 1, unroll=4)(fl_body)
    pltpu.sync_copy(out_v, wq_hbm.at[pl.ds(f_base, FPW)])


def _tc_matmul_body(x_ref, wq_ref, bias_ref, out_ref):
    out = lax.dot_general(
        x_ref[0], wq_ref[...], (((1,), (1,)), ((), ())),
        preferred_element_type=jnp.float32,
    )
    out_ref[0] = out + bias_ref[...]


def kernel(x, scale, bias, binary):
    size_out = x.shape[:-1] + (NF,)
    x3 = x.reshape(1, -1, NX)
    tt = x3.shape[1]
    scale_pad = jnp.concatenate(
        [scale.reshape(NBITS, NF).T,
         jnp.zeros((NF, 16 - NBITS), jnp.float32)], axis=1)

    sc_decode = functools.partial(
        pl.kernel,
        out_type=jax.ShapeDtypeStruct((NF, NX), jnp.float32),
        mesh=plsc.VectorSubcoreMesh(
            core_axis_name="c", subcore_axis_name="s",
            num_cores=NC, num_subcores=NS,
        ),
        compiler_params=pltpu.CompilerParams(
            needs_layout_passes=False, use_tc_tiling_on_sc=True),
        scratch_types=[
            pltpu.VMEM((NBITS, FPW, NX8), jnp.int32),
            pltpu.VMEM((FPW, 16), jnp.float32),
            pltpu.VMEM((FPW, NX), jnp.float32),
            pltpu.VMEM((FPW * 256,), jnp.float32),
        ],
    )(_sc_decode_body)
    wq = sc_decode(binary, scale_pad)

    out = pl.pallas_call(
        _tc_matmul_body,
        out_shape=jax.ShapeDtypeStruct((1, tt, NF), jnp.float32),
    )(x3, wq, bias.reshape(1, NF))
    return out.reshape(size_out)


# speedup vs baseline: 1.0227x; 1.0227x over previous
import functools
import jax
import jax.numpy as jnp
from jax import lax
from jax.experimental import pallas as pl
from jax.experimental.pallas import tpu as pltpu
from jax.experimental.pallas import tpu_sc as plsc

NX = 768
NF = 768
NX8 = NX // 8
NBITS = 8

NC, NS = 2, 16
NW = NC * NS
FPW = NF // NW
GV = NX8 // 16

_MAA = 0x00AA00AA
_MCC = 0x0000CCCC
_M0F = 0x0F0F0F0F
_MF0 = -252645136


def _sc_decode_body(codes_hbm, scale_hbm, wq_hbm,
                    codes_v, scale_v, out_v, slut_v):
    wid = lax.axis_index("s") * NC + lax.axis_index("c")
    f_base = wid * FPW
    pltpu.sync_copy(scale_hbm.at[pl.ds(f_base, FPW)], scale_v)
    pltpu.sync_copy(codes_hbm.at[:, pl.ds(f_base, FPW), :], codes_v)

    iota = lax.broadcasted_iota(jnp.int32, (16,), 0)
    iota8 = iota * 8

    def fl_body(fl):
        f_abs = f_base + fl
        svvec = scale_v[fl, :]
        sv = [jnp.full((16,), svvec[i], jnp.float32) for i in range(NBITS)]
        lo = jnp.zeros((16,), jnp.float32)
        hi = jnp.zeros((16,), jnp.float32)
        for j in range(4):
            bit = (iota >> (3 - j)) & 1
            lo = lo + jnp.where(bit != 0, sv[4 + j], -sv[4 + j])
            hi = hi + jnp.where(bit != 0, sv[j], -sv[j])
        sbase = fl * 256
        for k in range(16):
            slut_v[pl.ds(sbase + k * 16, 16)] = lo + hi[k]

        flvec = jnp.full((16,), fl, jnp.int32)
        for gh in range(GV // 2):
            rows2 = []
            for gv in (2 * gh, 2 * gh + 1):
                gsl = pl.ds(gv * 16, 16)
                v = [codes_v[i, fl, gsl] for i in range(NBITS)]
                xw = (v[0] << 24) | (v[1] << 16) | (v[2] << 8) | v[3]
                yw = (v[4] << 24) | (v[5] << 16) | (v[6] << 8) | v[7]
                t = (xw ^ (xw >> 7)) & _MAA
                xw = xw ^ t ^ (t << 7)
                t = (yw ^ (yw >> 7)) & _MAA
                yw = yw ^ t ^ (t << 7)
                t = (xw ^ (xw >> 14)) & _MCC
                xw = xw ^ t ^ (t << 14)
                t = (yw ^ (yw >> 14)) & _MCC
                yw = yw ^ t ^ (t << 14)
                t = (xw & jnp.int32(_MF0)) | ((yw >> 4) & _M0F)
                yw = ((xw << 4) & jnp.int32(_MF0)) | (yw & _M0F)
                xw = t
                rows2.append([(xw >> 24) & 0xFF, (xw >> 16) & 0xFF,
                              (xw >> 8) & 0xFF, xw & 0xFF,
                              (yw >> 24) & 0xFF, (yw >> 16) & 0xFF,
                              (yw >> 8) & 0xFF, yw & 0xFF])
            vals2 = [[plsc.load_gather(slut_v, [rows2[h][p] + sbase])
                      for p in range(8)] for h in range(2)]
            for h in range(2):
                for p in range(8):
                    cidx = iota8 + (128 * (2 * gh + h) + p)
                    plsc.store_scatter(out_v, [flvec, cidx], vals2[h][p])

    plsc.parallel_loop(0, FPW, 1, unroll=2)(fl_body)
    pltpu.sync_copy(out_v, wq_hbm.at[pl.ds(f_base, FPW)])


def _tc_matmul_body(x_ref, wq_ref, bias_ref, out_ref):
    out = lax.dot_general(
        x_ref[0], wq_ref[...], (((1,), (1,)), ((), ())),
        preferred_element_type=jnp.float32,
    )
    out_ref[0] = out + bias_ref[...]


def kernel(x, scale, bias, binary):
    size_out = x.shape[:-1] + (NF,)
    x3 = x.reshape(1, -1, NX)
    tt = x3.shape[1]
    scale_pad = jnp.concatenate(
        [scale.reshape(NBITS, NF).T,
         jnp.zeros((NF, 16 - NBITS), jnp.float32)], axis=1)

    sc_decode = functools.partial(
        pl.kernel,
        out_type=jax.ShapeDtypeStruct((NF, NX), jnp.float32),
        mesh=plsc.VectorSubcoreMesh(
            core_axis_name="c", subcore_axis_name="s",
            num_cores=NC, num_subcores=NS,
        ),
        compiler_params=pltpu.CompilerParams(
            needs_layout_passes=False, use_tc_tiling_on_sc=True),
        scratch_types=[
            pltpu.VMEM((NBITS, FPW, NX8), jnp.int32),
            pltpu.VMEM((FPW, 16), jnp.float32),
            pltpu.VMEM((FPW, NX), jnp.float32),
            pltpu.VMEM((FPW * 256,), jnp.float32),
        ],
    )(_sc_decode_body)
    wq = sc_decode(binary, scale_pad)

    out = pl.pallas_call(
        _tc_matmul_body,
        out_shape=jax.ShapeDtypeStruct((1, tt, NF), jnp.float32),
    )(x3, wq, bias.reshape(1, NF))
    return out.reshape(size_out)


# speedup vs baseline: 1.0269x; 1.0041x over previous
import functools
import jax
import jax.numpy as jnp
from jax import lax
from jax.experimental import pallas as pl
from jax.experimental.pallas import tpu as pltpu
from jax.experimental.pallas import tpu_sc as plsc

NX = 768
NF = 768
NX8 = NX // 8
NBITS = 8

NC, NS = 2, 16
NW = NC * NS
FPW = NF // NW
GV = NX8 // 16

_MAA = 0x00AA00AA
_MCC = 0x0000CCCC
_M0F = 0x0F0F0F0F
_MF0 = -252645136


def _sc_decode_body(codes_hbm, scale_hbm, wq_hbm,
                    codes_v, scale_v, out_v, slut_v):
    wid = lax.axis_index("s") * NC + lax.axis_index("c")
    f_base = wid * FPW
    pltpu.sync_copy(scale_hbm.at[pl.ds(f_base, FPW)], scale_v)
    pltpu.sync_copy(codes_hbm.at[:, pl.ds(f_base, FPW), :], codes_v)

    iota = lax.broadcasted_iota(jnp.int32, (16,), 0)
    iota8 = iota * 8

    def fl_body(fl):
        svvec = scale_v[fl, :]
        sv = [jnp.full((16,), svvec[i], jnp.float32) for i in range(NBITS)]
        lo = jnp.zeros((16,), jnp.float32)
        hi = jnp.zeros((16,), jnp.float32)
        for j in range(4):
            bit = (iota >> (3 - j)) & 1
            lo = lo + jnp.where(bit != 0, sv[4 + j], -sv[4 + j])
            hi = hi + jnp.where(bit != 0, sv[j], -sv[j])
        sbase = fl * 256
        for k in range(16):
            slut_v[pl.ds(sbase + k * 16, 16)] = lo + hi[k]

        flvec = jnp.full((16,), fl, jnp.int32)
        for gh in range(GV // 2):
            rows2 = []
            for gv in (2 * gh, 2 * gh + 1):
                gsl = pl.ds(gv * 16, 16)
                v = [codes_v[i, fl, gsl] for i in range(NBITS)]
                xw = (v[0] << 24) | (v[1] << 16) | (v[2] << 8) | v[3]
                yw = (v[4] << 24) | (v[5] << 16) | (v[6] << 8) | v[7]
                t = (xw ^ (xw >> 7)) & _MAA
                xw = xw ^ t ^ (t << 7)
                t = (yw ^ (yw >> 7)) & _MAA
                yw = yw ^ t ^ (t << 7)
                t = (xw ^ (xw >> 14)) & _MCC
                xw = xw ^ t ^ (t << 14)
                t = (yw ^ (yw >> 14)) & _MCC
                yw = yw ^ t ^ (t << 14)
                t = (xw & jnp.int32(_MF0)) | ((yw >> 4) & _M0F)
                yw = ((xw << 4) & jnp.int32(_MF0)) | (yw & _M0F)
                xw = t
                rows2.append([(xw >> 24) & 0xFF, (xw >> 16) & 0xFF,
                              (xw >> 8) & 0xFF, xw & 0xFF,
                              (yw >> 24) & 0xFF, (yw >> 16) & 0xFF,
                              (yw >> 8) & 0xFF, yw & 0xFF])
            vals2 = [[plsc.load_gather(slut_v, [rows2[h][p] + sbase])
                      for p in range(8)] for h in range(2)]
            for h in range(2):
                for p in range(8):
                    cidx = iota8 + (128 * (2 * gh + h) + p)
                    plsc.store_scatter(out_v, [flvec, cidx], vals2[h][p])

    plsc.parallel_loop(0, FPW, 1, unroll=2)(fl_body)
    pltpu.sync_copy(out_v, wq_hbm.at[pl.ds(f_base, FPW)])


def _tc_matmul_body(x_ref, wq_ref, bias_ref, out_ref):
    out = lax.dot_general(
        x_ref[0], wq_ref[...], (((1,), (1,)), ((), ())),
        preferred_element_type=jnp.float32,
    )
    out_ref[0] = out + bias_ref[...]


def kernel(x, scale, bias, binary):
    size_out = x.shape[:-1] + (NF,)
    x3 = x.reshape(1, -1, NX)
    tt = x3.shape[1]
    scale_pad = jnp.concatenate(
        [scale.reshape(NBITS, NF).T,
         jnp.zeros((NF, 16 - NBITS), jnp.float32)], axis=1)

    sc_decode = functools.partial(
        pl.kernel,
        out_type=jax.ShapeDtypeStruct((NF, NX), jnp.float32),
        mesh=plsc.VectorSubcoreMesh(
            core_axis_name="c", subcore_axis_name="s",
            num_cores=NC, num_subcores=NS,
        ),
        compiler_params=pltpu.CompilerParams(
            needs_layout_passes=False, use_tc_tiling_on_sc=True),
        scratch_types=[
            pltpu.VMEM((NBITS, FPW, NX8), jnp.int32),
            pltpu.VMEM((FPW, 16), jnp.float32),
            pltpu.VMEM((FPW, NX), jnp.float32),
            pltpu.VMEM((FPW * 256,), jnp.float32),
        ],
    )(_sc_decode_body)
    wq = sc_decode(binary, scale_pad)

    out = pl.pallas_call(
        _tc_matmul_body,
        out_shape=jax.ShapeDtypeStruct((1, tt, NF), jnp.float32),
    )(x3, wq, bias.reshape(1, NF))
    return out.reshape(size_out)
